# baseline (speedup 1.0000x reference)
import jax
import jax.numpy as jnp
from jax import lax
from jax.experimental import pallas as pl
from jax.experimental.pallas import tpu as pltpu
from jax.experimental.pallas import tpu_sc as plsc

_BATCH = 16384
_NF = 100
_D = 32
_NC = 2
_NS = 16
_NW = _NC * _NS
_RPW = _BATCH // _NW
_R = 2
_NCHUNK = _RPW // _R


def _sc_body(x_hbm, m_hbm, wv_hbm, bv_hbm, dt_hbm, mt_hbm, out_hbm,
             xb, mb, ob, wvt, bvt, dtt, mtt,
             sem_x, sem_m, sem_o):
    wid = lax.axis_index("s") * _NC + lax.axis_index("c")
    base = wid * _RPW

    pltpu.sync_copy(wv_hbm, wvt)
    pltpu.sync_copy(bv_hbm, bvt)
    pltpu.sync_copy(dt_hbm, dtt)
    pltpu.sync_copy(mt_hbm, mtt)
    wv0 = wvt[pl.ds(0, 16)]
    wv1 = wvt[pl.ds(16, 16)]
    bv0 = bvt[pl.ds(0, 16)]
    bv1 = bvt[pl.ds(16, 16)]

    def issue_in(ch, b):
        row0 = base + ch * _R
        pltpu.async_copy(x_hbm.at[pl.ds(row0, _R)], xb.at[b], sem_x.at[b])
        pltpu.async_copy(m_hbm.at[pl.ds(row0, _R)], mb.at[b], sem_m.at[b])

    issue_in(0, 0)
    issue_in(1, 1)

    def step(i, _):
        for b in range(2):
            ch = i * 2 + b
            row0 = base + ch * _R
            pltpu.make_async_copy(
                x_hbm.at[pl.ds(row0, _R)], xb.at[b], sem_x.at[b]).wait()
            pltpu.make_async_copy(
                m_hbm.at[pl.ds(row0, _R)], mb.at[b], sem_m.at[b]).wait()

            @pl.when(i >= 1)
            def _wait_out():
                pltpu.make_async_copy(
                    ob.at[b], out_hbm.at[pl.ds(row0, _R)], sem_o.at[b]).wait()

            def emit_feature(r, j, u, v):
                uvec = lax.broadcast(u, (16,))
                vvec = lax.broadcast(v, (16,))
                ob[b, r, j, pl.ds(0, 16)] = uvec * wv0 + vvec * bv0
                ob[b, r, j, pl.ds(16, 16)] = uvec * wv1 + vvec * bv1
                d0 = dtt[pl.ds(j * _D, 16)]
                m0 = mtt[pl.ds(j * _D, 16)]
                ob[b, r, j, pl.ds(32, 16)] = vvec * d0 + m0
                d1 = dtt[pl.ds(j * _D + 16, 16)]
                m1 = mtt[pl.ds(j * _D + 16, 16)]
                ob[b, r, j, pl.ds(48, 16)] = vvec * d1 + m1

            def rbody(r, carry):
                def jgbody(jg, c2):
                    xv = xb[b, r, pl.ds(jg * 16, 16)]
                    mv = mb[b, r, pl.ds(jg * 16, 16)]
                    vv = 1.0 - mv
                    uv = xv * vv
                    for t in range(16):
                        emit_feature(r, jg * 16 + t, uv[t], vv[t])
                    return c2
                lax.fori_loop(0, _NF // 16, jgbody, 0)
                xv = xb[b, r, pl.ds(_NF - 16, 16)]
                mv = mb[b, r, pl.ds(_NF - 16, 16)]
                vv = 1.0 - mv
                uv = xv * vv
                for t in range(16 - (_NF % 16), 16):
                    emit_feature(r, _NF - 16 + t, uv[t], vv[t])
                return carry
            lax.fori_loop(0, _R, rbody, 0)

            pltpu.async_copy(
                ob.at[b], out_hbm.at[pl.ds(row0, _R)], sem_o.at[b])

            @pl.when(i < _NCHUNK // 2 - 1)
            def _prefetch():
                issue_in(ch + 2, b)
        return 0

    lax.fori_loop(0, _NCHUNK // 2, step, 0)

    for b in range(2):
        row0 = base + (_NCHUNK - 2 + b) * _R
        pltpu.make_async_copy(
            ob.at[b], out_hbm.at[pl.ds(row0, _R)], sem_o.at[b]).wait()


@jax.jit
def kernel(x_hat, mask, Wv, bv, missing_table, present_table):
    wv = Wv[:, 0]
    dt = (present_table - missing_table).reshape(_NF * _D)
    mt = missing_table.reshape(_NF * _D)
    mesh = plsc.VectorSubcoreMesh(core_axis_name="c", subcore_axis_name="s")
    run = pl.kernel(
        _sc_body,
        out_type=jax.ShapeDtypeStruct((_BATCH, _NF, 2 * _D), jnp.float32),
        mesh=mesh,
        scratch_types=[
            pltpu.VMEM((2, _R, _NF), jnp.float32),
            pltpu.VMEM((2, _R, _NF), jnp.float32),
            pltpu.VMEM((2, _R, _NF, 2 * _D), jnp.float32),
            pltpu.VMEM((_D,), jnp.float32),
            pltpu.VMEM((_D,), jnp.float32),
            pltpu.VMEM((_NF * _D,), jnp.float32),
            pltpu.VMEM((_NF * _D,), jnp.float32),
            pltpu.SemaphoreType.DMA((2,)),
            pltpu.SemaphoreType.DMA((2,)),
            pltpu.SemaphoreType.DMA((2,)),
        ],
    )
    return run(x_hat, mask, wv, bv, dt, mt)


# speedup vs baseline: 1.3060x; 1.3060x over previous
import jax
import jax.numpy as jnp
from jax import lax
from jax.experimental import pallas as pl
from jax.experimental.pallas import tpu as pltpu
from jax.experimental.pallas import tpu_sc as plsc

_BATCH = 16384
_NF = 100
_D = 32
_NC = 2
_NS = 16
_NW = _NC * _NS
_RPW = _BATCH // _NW
_R = 2
_NCHUNK = _RPW // _R


def _sc_body(x_hbm, m_hbm, wv_hbm, bv_hbm, dt_hbm, mt_hbm, out_hbm,
             xb, mb, ob, wvt, bvt, dtt, mtt,
             sem_x, sem_m, sem_o):
    wid = lax.axis_index("s") * _NC + lax.axis_index("c")
    base = wid * _RPW

    pltpu.sync_copy(wv_hbm, wvt)
    pltpu.sync_copy(bv_hbm, bvt)
    pltpu.sync_copy(dt_hbm, dtt)
    pltpu.sync_copy(mt_hbm, mtt)
    wv0 = wvt[pl.ds(0, 16)]
    wv1 = wvt[pl.ds(16, 16)]
    bv0 = bvt[pl.ds(0, 16)]
    bv1 = bvt[pl.ds(16, 16)]

    def issue_in(ch, b):
        row0 = base + ch * _R
        pltpu.async_copy(x_hbm.at[pl.ds(row0, _R)], xb.at[b], sem_x.at[b])
        pltpu.async_copy(m_hbm.at[pl.ds(row0, _R)], mb.at[b], sem_m.at[b])

    issue_in(0, 0)
    issue_in(1, 1)

    def step(i, _):
        for b in range(2):
            ch = i * 2 + b
            row0 = base + ch * _R
            pltpu.make_async_copy(
                x_hbm.at[pl.ds(row0, _R)], xb.at[b], sem_x.at[b]).wait()
            pltpu.make_async_copy(
                m_hbm.at[pl.ds(row0, _R)], mb.at[b], sem_m.at[b]).wait()

            @pl.when(i >= 1)
            def _wait_out():
                pltpu.make_async_copy(
                    ob.at[b], out_hbm.at[pl.ds(row0, _R)], sem_o.at[b]).wait()

            def emit_feature(r, j, u, v):
                uvec = lax.broadcast(u, (16,))
                vvec = lax.broadcast(v, (16,))
                ob[b, r, j, pl.ds(0, 16)] = uvec * wv0 + vvec * bv0
                ob[b, r, j, pl.ds(16, 16)] = uvec * wv1 + vvec * bv1
                d0 = dtt[pl.ds(j * _D, 16)]
                m0 = mtt[pl.ds(j * _D, 16)]
                ob[b, r, j, pl.ds(32, 16)] = vvec * d0 + m0
                d1 = dtt[pl.ds(j * _D + 16, 16)]
                m1 = mtt[pl.ds(j * _D + 16, 16)]
                ob[b, r, j, pl.ds(48, 16)] = vvec * d1 + m1

            for r in range(_R):
                @plsc.parallel_loop(0, _NF // 16, unroll=2)
                def jgbody(jg):
                    xv = xb[b, r, pl.ds(jg * 16, 16)]
                    mv = mb[b, r, pl.ds(jg * 16, 16)]
                    vv = 1.0 - mv
                    uv = xv * vv
                    for t in range(16):
                        emit_feature(r, jg * 16 + t, uv[t], vv[t])
                xv = xb[b, r, pl.ds(_NF - 16, 16)]
                mv = mb[b, r, pl.ds(_NF - 16, 16)]
                vv = 1.0 - mv
                uv = xv * vv
                for t in range(16 - (_NF % 16), 16):
                    emit_feature(r, _NF - 16 + t, uv[t], vv[t])

            pltpu.async_copy(
                ob.at[b], out_hbm.at[pl.ds(row0, _R)], sem_o.at[b])

            @pl.when(i < _NCHUNK // 2 - 1)
            def _prefetch():
                issue_in(ch + 2, b)
        return 0

    lax.fori_loop(0, _NCHUNK // 2, step, 0)

    for b in range(2):
        row0 = base + (_NCHUNK - 2 + b) * _R
        pltpu.make_async_copy(
            ob.at[b], out_hbm.at[pl.ds(row0, _R)], sem_o.at[b]).wait()


@jax.jit
def kernel(x_hat, mask, Wv, bv, missing_table, present_table):
    wv = Wv[:, 0]
    dt = (present_table - missing_table).reshape(_NF * _D)
    mt = missing_table.reshape(_NF * _D)
    mesh = plsc.VectorSubcoreMesh(core_axis_name="c", subcore_axis_name="s")
    run = pl.kernel(
        _sc_body,
        out_type=jax.ShapeDtypeStruct((_BATCH, _NF, 2 * _D), jnp.float32),
        mesh=mesh,
        scratch_types=[
            pltpu.VMEM((2, _R, _NF), jnp.float32),
            pltpu.VMEM((2, _R, _NF), jnp.float32),
            pltpu.VMEM((2, _R, _NF, 2 * _D), jnp.float32),
            pltpu.VMEM((_D,), jnp.float32),
            pltpu.VMEM((_D,), jnp.float32),
            pltpu.VMEM((_NF * _D,), jnp.float32),
            pltpu.VMEM((_NF * _D,), jnp.float32),
            pltpu.SemaphoreType.DMA((2,)),
            pltpu.SemaphoreType.DMA((2,)),
            pltpu.SemaphoreType.DMA((2,)),
        ],
    )
    return run(x_hat, mask, wv, bv, dt, mt)


# speedup vs baseline: 6.8535x; 5.2477x over previous
import jax
import jax.numpy as jnp
from jax.experimental import pallas as pl

_BATCH = 16384
_NF = 100
_D = 32
_JBLK = 2


def _body(x_ref, m_ref, a_ref, b_ref, c_ref, o_ref):
    v = 1.0 - m_ref[...]
    u = x_ref[...] * v
    shape = (_JBLK, 2 * _D, _BATCH)
    ub = jnp.broadcast_to(u, shape)
    vb = jnp.broadcast_to(v, shape)
    ab = jnp.broadcast_to(a_ref[...], shape)
    bb = jnp.broadcast_to(b_ref[...], shape)
    cb = jnp.broadcast_to(c_ref[...], shape)
    o_ref[...] = ub * ab + (vb * bb + cb)


@jax.jit
def kernel(x_hat, mask, Wv, bv, missing_table, present_table):
    wv = Wv[:, 0]
    a_t = jnp.concatenate(
        [jnp.broadcast_to(wv, (_NF, _D)), jnp.zeros((_NF, _D), jnp.float32)],
        axis=1,
    ).reshape(_NF, 2 * _D, 1)
    b_t = jnp.concatenate(
        [jnp.broadcast_to(bv, (_NF, _D)), present_table - missing_table],
        axis=1,
    ).reshape(_NF, 2 * _D, 1)
    c_t = jnp.concatenate(
        [jnp.zeros((_NF, _D), jnp.float32), missing_table], axis=1
    ).reshape(_NF, 2 * _D, 1)
    x_t = x_hat.T.reshape(_NF, 1, _BATCH)
    m_t = mask.T.reshape(_NF, 1, _BATCH)
    grid = (_NF // _JBLK,)
    out_t = pl.pallas_call(
        _body,
        grid=grid,
        in_specs=[
            pl.BlockSpec((_JBLK, 1, _BATCH), lambda i: (i, 0, 0)),
            pl.BlockSpec((_JBLK, 1, _BATCH), lambda i: (i, 0, 0)),
            pl.BlockSpec((_JBLK, 2 * _D, 1), lambda i: (i, 0, 0)),
            pl.BlockSpec((_JBLK, 2 * _D, 1), lambda i: (i, 0, 0)),
            pl.BlockSpec((_JBLK, 2 * _D, 1), lambda i: (i, 0, 0)),
        ],
        out_specs=pl.BlockSpec((_JBLK, 2 * _D, _BATCH), lambda i: (i, 0, 0)),
        out_shape=jax.ShapeDtypeStruct((_NF, 2 * _D, _BATCH), jnp.float32),
    )(x_t, m_t, a_t, b_t, c_t)
    return jnp.transpose(out_t, (2, 0, 1))
